# Initial kernel scaffold; baseline (speedup 1.0000x reference)
import functools

import jax
import jax.numpy as jnp
from jax import lax
from jax.experimental import pallas as pl
from jax.experimental.pallas import tpu as pltpu
from jax.experimental.pallas import tpu_sc as plsc

_PIVOT = 500000
_D = 64
_L = 16
_NW = 32
_N = 16384 * 50
_PER_W = _N // _NW
_CHUNK = 128
_NCHUNK = _PER_W // _CHUNK
_NBLK = _PER_W // _L


def _body(x_hbm, w1_hbm, w2_hbm, out_hbm,
          x_v, idx_v, pos_v, pos_c, buf_a, buf_b, sem_g, sem_s):
    wid = lax.axis_index("s") * 2 + lax.axis_index("c")
    base = wid * _PER_W
    pltpu.sync_copy(x_hbm.at[pl.ds(base, _PER_W)], x_v)

    lane = lax.iota(jnp.int32, _L)

    def compact_step(i, carry):
        o1, o2 = carry
        xv = x_v[pl.ds(i * _L, _L)]
        m1 = xv < _PIVOT
        iv = jnp.where(m1, xv, xv - _PIVOT)
        pv = base + i * _L + lane
        c1 = jnp.sum(m1.astype(jnp.int32))
        o2n = o2 - (_L - c1)
        plsc.store_compressed(idx_v.at[pl.ds(o1, _L)], iv, mask=m1)
        plsc.store_compressed(pos_v.at[pl.ds(o1, _L)], pv, mask=m1)
        m2 = jnp.logical_not(m1)
        plsc.store_compressed(idx_v.at[pl.ds(o2n, _L)], iv, mask=m2)
        plsc.store_compressed(pos_v.at[pl.ds(o2n, _L)], pv, mask=m2)
        return o1 + c1, o2n

    k1, _ = lax.fori_loop(0, _NBLK, compact_step,
                          (jnp.int32(0), jnp.int32(_PER_W)))

    def chunk_step(c, carry):
        r = k1 - c * _CHUNK
        idx_slice = idx_v.at[pl.ds(c * _CHUNK, _CHUNK)]

        @pl.when(r >= _CHUNK)
        def _():
            pltpu.async_copy(w1_hbm.at[idx_slice], buf_a, sem_g).wait()

        @pl.when(r <= 0)
        def _():
            pltpu.async_copy(w2_hbm.at[idx_slice], buf_a, sem_g).wait()

        @pl.when(jnp.logical_and(r > 0, r < _CHUNK))
        def _():
            pltpu.async_copy(w1_hbm.at[idx_slice], buf_a, sem_g).wait()
            pltpu.async_copy(w2_hbm.at[idx_slice], buf_b, sem_g).wait()

            def merge_row(j, c2):
                @pl.when(j >= r)
                def _():
                    for d in range(_D // _L):
                        buf_a[j, pl.ds(d * _L, _L)] = buf_b[j, pl.ds(d * _L, _L)]
                return c2

            lax.fori_loop(0, _CHUNK, merge_row, 0)

        pltpu.sync_copy(pos_v.at[pl.ds(c * _CHUNK, _CHUNK)], pos_c.at[0])
        pltpu.async_copy(buf_a, out_hbm.at[pos_c.at[0]], sem_s).wait()
        return carry

    lax.fori_loop(0, _NCHUNK, chunk_step, 0)


_lookup = functools.partial(
    pl.kernel,
    out_type=jax.ShapeDtypeStruct((_N, _D), jnp.float32),
    mesh=plsc.VectorSubcoreMesh(core_axis_name="c", subcore_axis_name="s"),
    scratch_types=[
        pltpu.VMEM((_PER_W,), jnp.int32),
        pltpu.VMEM((_PER_W + _L,), jnp.int32),
        pltpu.VMEM((_PER_W + _L,), jnp.int32),
        pltpu.VMEM((1, _CHUNK), jnp.int32),
        pltpu.VMEM((_CHUNK, _D), jnp.float32),
        pltpu.VMEM((_CHUNK, _D), jnp.float32),
        pltpu.SemaphoreType.DMA,
        pltpu.SemaphoreType.DMA,
    ],
)(_body)


def kernel(x, W1, W2):
    xf = x.reshape(-1).astype(jnp.int32)
    out = _lookup(xf, W1, W2)
    return out.reshape(x.shape + (_D,))


# speedup vs baseline: 25.4454x; 25.4454x over previous
import functools

import jax
import jax.numpy as jnp
from jax import lax
from jax.experimental import pallas as pl
from jax.experimental.pallas import tpu as pltpu
from jax.experimental.pallas import tpu_sc as plsc

_PIVOT = 500000
_D = 64
_L = 16
_NW = 32
_N = 16384 * 50
_PER_W = _N // _NW
_CHUNK = 128
_NCHUNK = _PER_W // _CHUNK
_NBLK = _PER_W // _L


def _body(x_hbm, w1_hbm, w2_hbm, out_hbm,
          x_v, idx_v, pos_v, pos_c, buf_a, buf_b, cnt_v, sem_g, sem_s):
    wid = lax.axis_index("s") * 2 + lax.axis_index("c")
    base = wid * _PER_W
    pltpu.sync_copy(x_hbm.at[pl.ds(base, _PER_W)], x_v)

    lane = lax.iota(jnp.int32, _L)

    def compact_step(i, carry):
        o1, o2 = carry
        xv = x_v[pl.ds(i * _L, _L)]
        m1 = xv < _PIVOT
        iv = jnp.where(m1, xv, xv - _PIVOT)
        pv = base + i * _L + lane
        cv = jnp.where(m1, jnp.int32(1), jnp.int32(0))
        for h in (8, 4, 2, 1):
            cnt_v[pl.ds(0, _L)] = cv
            cv = cv + plsc.load_gather(cnt_v, [lane ^ h])
        c1 = cv[0]
        o2n = o2 - (_L - c1)
        plsc.store_compressed(idx_v.at[pl.ds(o1, _L)], iv, mask=m1)
        plsc.store_compressed(pos_v.at[pl.ds(o1, _L)], pv, mask=m1)
        m2 = jnp.logical_not(m1)
        plsc.store_compressed(idx_v.at[pl.ds(o2n, _L)], iv, mask=m2)
        plsc.store_compressed(pos_v.at[pl.ds(o2n, _L)], pv, mask=m2)
        return o1 + c1, o2n

    k1, _ = lax.fori_loop(0, _NBLK, compact_step,
                          (jnp.int32(0), jnp.int32(_PER_W)))

    def chunk_step(c, carry):
        r = k1 - c * _CHUNK
        idx_slice = idx_v.at[pl.ds(c * _CHUNK, _CHUNK)]

        @pl.when(r >= _CHUNK)
        def _():
            pltpu.async_copy(w1_hbm.at[idx_slice], buf_a, sem_g).wait()

        @pl.when(r <= 0)
        def _():
            pltpu.async_copy(w2_hbm.at[idx_slice], buf_a, sem_g).wait()

        @pl.when(jnp.logical_and(r > 0, r < _CHUNK))
        def _():
            pltpu.async_copy(w1_hbm.at[idx_slice], buf_a, sem_g).wait()
            pltpu.async_copy(w2_hbm.at[idx_slice], buf_b, sem_g).wait()

            def merge_row(j, c2):
                @pl.when(j >= r)
                def _():
                    for d in range(_D // _L):
                        buf_a[j, pl.ds(d * _L, _L)] = buf_b[j, pl.ds(d * _L, _L)]
                return c2

            lax.fori_loop(0, _CHUNK, merge_row, 0)

        for t in range(_CHUNK // _L):
            pos_c[0, pl.ds(t * _L, _L)] = pos_v[pl.ds(c * _CHUNK + t * _L, _L)]
        pltpu.async_copy(buf_a, out_hbm.at[pos_c.at[0]], sem_s).wait()
        return carry

    lax.fori_loop(0, _NCHUNK, chunk_step, 0)


_lookup = functools.partial(
    pl.kernel,
    out_type=jax.ShapeDtypeStruct((_N, _D), jnp.float32),
    mesh=plsc.VectorSubcoreMesh(core_axis_name="c", subcore_axis_name="s"),
    compiler_params=pltpu.CompilerParams(needs_layout_passes=False, use_tc_tiling_on_sc=False),
    scratch_types=[
        pltpu.VMEM((_PER_W,), jnp.int32),
        pltpu.VMEM((_PER_W + _L,), jnp.int32),
        pltpu.VMEM((_PER_W + _L,), jnp.int32),
        pltpu.VMEM((1, _CHUNK), jnp.int32),
        pltpu.VMEM((_CHUNK, _D), jnp.float32),
        pltpu.VMEM((_CHUNK, _D), jnp.float32),
        pltpu.VMEM((_L,), jnp.int32),
        pltpu.SemaphoreType.DMA,
        pltpu.SemaphoreType.DMA,
    ],
)(_body)


def kernel(x, W1, W2):
    xf = x.reshape(-1).astype(jnp.int32)
    out = _lookup(xf, W1, W2)
    return out.reshape(x.shape + (_D,))


# speedup vs baseline: 28.5087x; 1.1204x over previous
import functools

import jax
import jax.numpy as jnp
from jax import lax
from jax.experimental import pallas as pl
from jax.experimental.pallas import tpu as pltpu
from jax.experimental.pallas import tpu_sc as plsc

_PIVOT = 500000
_D = 64
_L = 16
_NW = 32
_N = 16384 * 50
_PER_W = _N // _NW
_CHUNK = 128
_NCHUNK = _PER_W // _CHUNK
_NBLK = _PER_W // _L
_NBUF = 4


def _body(x_hbm, w1_hbm, w2_hbm, out_hbm,
          x_v, idx_v, pos_v, pos_c, b0, b1, b2, b3, buf_b, cnt_v,
          sg0, sg1, sg2, sg3, ss0, ss1, ss2, ss3, sem_gb):
    bufs = (b0, b1, b2, b3)
    sems_g = (sg0, sg1, sg2, sg3)
    sems_s = (ss0, ss1, ss2, ss3)

    wid = lax.axis_index("s") * 2 + lax.axis_index("c")
    base = wid * _PER_W
    pltpu.sync_copy(x_hbm.at[pl.ds(base, _PER_W)], x_v)

    lane = lax.iota(jnp.int32, _L)

    def compact_step(i, carry):
        o1, o2 = carry
        xv = x_v[pl.ds(i * _L, _L)]
        m1 = xv < _PIVOT
        iv = jnp.where(m1, xv, xv - _PIVOT)
        pv = base + i * _L + lane
        cv = jnp.where(m1, jnp.int32(1), jnp.int32(0))
        for h in (8, 4, 2, 1):
            cnt_v[pl.ds(0, _L)] = cv
            cv = cv + plsc.load_gather(cnt_v, [lane ^ h])
        c1 = cv[0]
        o2n = o2 - (_L - c1)
        plsc.store_compressed(idx_v.at[pl.ds(o1, _L)], iv, mask=m1)
        plsc.store_compressed(pos_v.at[pl.ds(o1, _L)], pv, mask=m1)
        m2 = jnp.logical_not(m1)
        plsc.store_compressed(idx_v.at[pl.ds(o2n, _L)], iv, mask=m2)
        plsc.store_compressed(pos_v.at[pl.ds(o2n, _L)], pv, mask=m2)
        return o1 + c1, o2n

    k1, _ = lax.fori_loop(0, _NBLK, compact_step,
                          (jnp.int32(0), jnp.int32(_PER_W)))

    def idx_slice(c):
        return idx_v.at[pl.ds(c * _CHUNK, _CHUNK)]

    def issue_gather(c, p):
        r = k1 - c * _CHUNK
        sl = idx_slice(c)

        @pl.when(r >= _CHUNK)
        def _():
            pltpu.async_copy(w1_hbm.at[sl], bufs[p], sems_g[p])

        @pl.when(r <= 0)
        def _():
            pltpu.async_copy(w2_hbm.at[sl], bufs[p], sems_g[p])

        @pl.when(jnp.logical_and(r > 0, r < _CHUNK))
        def _():
            pltpu.async_copy(w1_hbm.at[sl], bufs[p], sems_g[p])
            pltpu.async_copy(w2_hbm.at[sl], buf_b, sem_gb)

    def wait_gather(c, p):
        r = k1 - c * _CHUNK
        pltpu.make_async_copy(w1_hbm.at[idx_slice(c)], bufs[p], sems_g[p]).wait()

        @pl.when(jnp.logical_and(r > 0, r < _CHUNK))
        def _():
            pltpu.make_async_copy(w2_hbm.at[idx_slice(c)], buf_b, sem_gb).wait()

            def merge_row(j, cc):
                @pl.when(j >= r)
                def _():
                    for d in range(_D // _L):
                        bufs[p][j, pl.ds(d * _L, _L)] = buf_b[j, pl.ds(d * _L, _L)]
                return cc

            lax.fori_loop(0, _CHUNK, merge_row, 0)

    def issue_scatter(c, p):
        for t in range(_CHUNK // _L):
            pos_c[p, pl.ds(t * _L, _L)] = pos_v[pl.ds(c * _CHUNK + t * _L, _L)]
        pltpu.async_copy(bufs[p], out_hbm.at[pos_c.at[p]], sems_s[p])

    def wait_scatter(c, p):
        pltpu.make_async_copy(bufs[p], out_hbm.at[pos_c.at[p]], sems_s[p]).wait()

    for u in range(_NBUF - 1):
        issue_gather(jnp.int32(u), u)

    def super_step(c4, carry):
        for u in range(_NBUF):
            t = c4 * _NBUF + u
            pw = (u + _NBUF - 1) % _NBUF

            if u == 0:
                @pl.when(c4 >= 1)
                def _():
                    wait_scatter(t - 1, pw)
            else:
                wait_scatter(t - 1, pw)

            @pl.when(t + _NBUF - 1 < _NCHUNK)
            def _():
                issue_gather(t + _NBUF - 1, pw)

            wait_gather(t, u)
            issue_scatter(t, u)
        return carry

    lax.fori_loop(0, _NCHUNK // _NBUF, super_step, 0)
    wait_scatter(_NCHUNK - 1, (_NCHUNK - 1) % _NBUF)


_lookup = functools.partial(
    pl.kernel,
    out_type=jax.ShapeDtypeStruct((_N, _D), jnp.float32),
    mesh=plsc.VectorSubcoreMesh(core_axis_name="c", subcore_axis_name="s"),
    compiler_params=pltpu.CompilerParams(needs_layout_passes=False, use_tc_tiling_on_sc=False),
    scratch_types=[
        pltpu.VMEM((_PER_W,), jnp.int32),
        pltpu.VMEM((_PER_W + _L,), jnp.int32),
        pltpu.VMEM((_PER_W + _L,), jnp.int32),
        pltpu.VMEM((_NBUF, _CHUNK), jnp.int32),
        pltpu.VMEM((_CHUNK, _D), jnp.float32),
        pltpu.VMEM((_CHUNK, _D), jnp.float32),
        pltpu.VMEM((_CHUNK, _D), jnp.float32),
        pltpu.VMEM((_CHUNK, _D), jnp.float32),
        pltpu.VMEM((_CHUNK, _D), jnp.float32),
        pltpu.VMEM((_L,), jnp.int32),
        pltpu.SemaphoreType.DMA,
        pltpu.SemaphoreType.DMA,
        pltpu.SemaphoreType.DMA,
        pltpu.SemaphoreType.DMA,
        pltpu.SemaphoreType.DMA,
        pltpu.SemaphoreType.DMA,
        pltpu.SemaphoreType.DMA,
        pltpu.SemaphoreType.DMA,
        pltpu.SemaphoreType.DMA,
    ],
)(_body)


def kernel(x, W1, W2):
    xf = x.reshape(-1).astype(jnp.int32)
    out = _lookup(xf, W1, W2)
    return out.reshape(x.shape + (_D,))


# speedup vs baseline: 29.3295x; 1.0288x over previous
import functools

import jax
import jax.numpy as jnp
from jax import lax
from jax.experimental import pallas as pl
from jax.experimental.pallas import tpu as pltpu
from jax.experimental.pallas import tpu_sc as plsc

_PIVOT = 500000
_D = 64
_L = 16
_NW = 32
_N = 16384 * 50
_PER_W = _N // _NW
_CHUNK = 128
_NCHUNK = _PER_W // _CHUNK
_NBLK = _PER_W // _L
_NBUF = 4


def _body(x_hbm, w1_hbm, w2_hbm, out_hbm,
          x_v, idx_v, pos_v, pos_c, b0, b1, b2, b3, buf_b, cnt_v,
          sg0, sg1, sg2, sg3, ss0, ss1, ss2, ss3, sem_gb):
    bufs = (b0, b1, b2, b3)
    sems_g = (sg0, sg1, sg2, sg3)
    sems_s = (ss0, ss1, ss2, ss3)

    wid = lax.axis_index("s") * 2 + lax.axis_index("c")
    base = wid * _PER_W
    pltpu.sync_copy(x_hbm.at[pl.ds(base, _PER_W)], x_v)

    lane = lax.iota(jnp.int32, _L)
    ones = jnp.ones((_L,), jnp.int32)

    def compact_step(i, carry):
        o1, o2 = carry
        xv = x_v[pl.ds(i * _L, _L)]
        m1 = xv < _PIVOT
        iv = jnp.where(m1, xv, xv - _PIVOT)
        pv = base + i * _L + lane
        c1v = plsc.all_reduce_population_count(m1)
        c1 = c1v[0]
        s1 = plsc.cumsum(jnp.where(m1, jnp.int32(1), jnp.int32(0)))
        dest = jnp.where(m1, o1 + s1 - 1, o2 - (lane + 1 - s1))
        plsc.store_scatter(idx_v, [dest], iv)
        plsc.store_scatter(pos_v, [dest], pv)
        return o1 + c1, o2 - (_L - c1)

    k1, _ = lax.fori_loop(0, _NBLK, compact_step,
                          (jnp.int32(0), jnp.int32(_PER_W)))

    def idx_slice(c):
        return idx_v.at[pl.ds(c * _CHUNK, _CHUNK)]

    def issue_gather(c, p):
        r = k1 - c * _CHUNK
        sl = idx_slice(c)

        @pl.when(r >= _CHUNK)
        def _():
            pltpu.async_copy(w1_hbm.at[sl], bufs[p], sems_g[p])

        @pl.when(r <= 0)
        def _():
            pltpu.async_copy(w2_hbm.at[sl], bufs[p], sems_g[p])

        @pl.when(jnp.logical_and(r > 0, r < _CHUNK))
        def _():
            pltpu.async_copy(w1_hbm.at[sl], bufs[p], sems_g[p])
            pltpu.async_copy(w2_hbm.at[sl], buf_b, sem_gb)

    def wait_gather(c, p):
        r = k1 - c * _CHUNK
        pltpu.make_async_copy(w1_hbm.at[idx_slice(c)], bufs[p], sems_g[p]).wait()

        @pl.when(jnp.logical_and(r > 0, r < _CHUNK))
        def _():
            pltpu.make_async_copy(w2_hbm.at[idx_slice(c)], buf_b, sem_gb).wait()

            def merge_row(j, cc):
                @pl.when(j >= r)
                def _():
                    for d in range(_D // _L):
                        bufs[p][j, pl.ds(d * _L, _L)] = buf_b[j, pl.ds(d * _L, _L)]
                return cc

            lax.fori_loop(0, _CHUNK, merge_row, 0)

    def issue_scatter(c, p):
        for t in range(_CHUNK // _L):
            pos_c[p, pl.ds(t * _L, _L)] = pos_v[pl.ds(c * _CHUNK + t * _L, _L)]
        pltpu.async_copy(bufs[p], out_hbm.at[pos_c.at[p]], sems_s[p])

    def wait_scatter(c, p):
        pltpu.make_async_copy(bufs[p], out_hbm.at[pos_c.at[p]], sems_s[p]).wait()

    for u in range(_NBUF - 1):
        issue_gather(jnp.int32(u), u)

    def super_step(c4, carry):
        for u in range(_NBUF):
            t = c4 * _NBUF + u
            pw = (u + _NBUF - 1) % _NBUF

            if u == 0:
                @pl.when(c4 >= 1)
                def _():
                    wait_scatter(t - 1, pw)
            else:
                wait_scatter(t - 1, pw)

            @pl.when(t + _NBUF - 1 < _NCHUNK)
            def _():
                issue_gather(t + _NBUF - 1, pw)

            wait_gather(t, u)
            issue_scatter(t, u)
        return carry

    lax.fori_loop(0, _NCHUNK // _NBUF, super_step, 0)
    wait_scatter(_NCHUNK - 1, (_NCHUNK - 1) % _NBUF)


_lookup = functools.partial(
    pl.kernel,
    out_type=jax.ShapeDtypeStruct((_N, _D), jnp.float32),
    mesh=plsc.VectorSubcoreMesh(core_axis_name="c", subcore_axis_name="s"),
    compiler_params=pltpu.CompilerParams(needs_layout_passes=False, use_tc_tiling_on_sc=False),
    scratch_types=[
        pltpu.VMEM((_PER_W,), jnp.int32),
        pltpu.VMEM((_PER_W + _L,), jnp.int32),
        pltpu.VMEM((_PER_W + _L,), jnp.int32),
        pltpu.VMEM((_NBUF, _CHUNK), jnp.int32),
        pltpu.VMEM((_CHUNK, _D), jnp.float32),
        pltpu.VMEM((_CHUNK, _D), jnp.float32),
        pltpu.VMEM((_CHUNK, _D), jnp.float32),
        pltpu.VMEM((_CHUNK, _D), jnp.float32),
        pltpu.VMEM((_CHUNK, _D), jnp.float32),
        pltpu.VMEM((_L,), jnp.int32),
        pltpu.SemaphoreType.DMA,
        pltpu.SemaphoreType.DMA,
        pltpu.SemaphoreType.DMA,
        pltpu.SemaphoreType.DMA,
        pltpu.SemaphoreType.DMA,
        pltpu.SemaphoreType.DMA,
        pltpu.SemaphoreType.DMA,
        pltpu.SemaphoreType.DMA,
        pltpu.SemaphoreType.DMA,
    ],
)(_body)


def kernel(x, W1, W2):
    xf = x.reshape(-1).astype(jnp.int32)
    out = _lookup(xf, W1, W2)
    return out.reshape(x.shape + (_D,))


# speedup vs baseline: 29.7365x; 1.0139x over previous
import functools

import jax
import jax.numpy as jnp
from jax import lax
from jax.experimental import pallas as pl
from jax.experimental.pallas import tpu as pltpu
from jax.experimental.pallas import tpu_sc as plsc

_PIVOT = 500000
_D = 64
_L = 16
_NW = 32
_N = 16384 * 50
_PER_W = _N // _NW
_CHUNK = 128
_NCHUNK = _PER_W // _CHUNK
_NBLK = _PER_W // _L
_NBUF = 4


def _body(x_hbm, w1_hbm, w2_hbm, out_hbm,
          x_v, idx_v, pos_v, pos_c, b0, b1, b2, b3, buf_b, cnt_v,
          sg0, sg1, sg2, sg3, ss0, ss1, ss2, ss3, sem_gb):
    bufs = (b0, b1, b2, b3)
    sems_g = (sg0, sg1, sg2, sg3)
    sems_s = (ss0, ss1, ss2, ss3)

    wid = lax.axis_index("s") * 2 + lax.axis_index("c")
    base = wid * _PER_W
    pltpu.sync_copy(x_hbm.at[pl.ds(base, _PER_W)], x_v)

    lane = lax.iota(jnp.int32, _L)
    ones = jnp.ones((_L,), jnp.int32)

    def compact_step(i, carry):
        o1v, o2v = carry
        xv = x_v[pl.ds(i * _L, _L)]
        m1 = xv < _PIVOT
        iv = jnp.where(m1, xv, xv - _PIVOT)
        pv = base + i * _L + lane
        c1v = plsc.all_reduce_population_count(m1)
        s1 = plsc.cumsum(jnp.where(m1, jnp.int32(1), jnp.int32(0)))
        dest = jnp.where(m1, o1v + s1 - 1, o2v - (lane + 1 - s1))
        plsc.store_scatter(idx_v, [dest], iv)
        plsc.store_scatter(pos_v, [dest], pv)
        return o1v + c1v, o2v - (_L - c1v)

    o1f, _ = plsc.parallel_loop(
        0, _NBLK, unroll=8,
        carry=(jnp.zeros((_L,), jnp.int32),
               jnp.full((_L,), _PER_W, jnp.int32)))(compact_step)
    k1 = o1f[0]

    def idx_slice(c):
        return idx_v.at[pl.ds(c * _CHUNK, _CHUNK)]

    def issue_gather(c, p):
        r = k1 - c * _CHUNK
        sl = idx_slice(c)

        @pl.when(r >= _CHUNK)
        def _():
            pltpu.async_copy(w1_hbm.at[sl], bufs[p], sems_g[p])

        @pl.when(r <= 0)
        def _():
            pltpu.async_copy(w2_hbm.at[sl], bufs[p], sems_g[p])

        @pl.when(jnp.logical_and(r > 0, r < _CHUNK))
        def _():
            pltpu.async_copy(w1_hbm.at[sl], bufs[p], sems_g[p])
            pltpu.async_copy(w2_hbm.at[sl], buf_b, sem_gb)

    def wait_gather(c, p):
        r = k1 - c * _CHUNK
        pltpu.make_async_copy(w1_hbm.at[idx_slice(c)], bufs[p], sems_g[p]).wait()

        @pl.when(jnp.logical_and(r > 0, r < _CHUNK))
        def _():
            pltpu.make_async_copy(w2_hbm.at[idx_slice(c)], buf_b, sem_gb).wait()

            def merge_row(j, cc):
                @pl.when(j >= r)
                def _():
                    for d in range(_D // _L):
                        bufs[p][j, pl.ds(d * _L, _L)] = buf_b[j, pl.ds(d * _L, _L)]
                return cc

            lax.fori_loop(0, _CHUNK, merge_row, 0)

    def issue_scatter(c, p):
        for t in range(_CHUNK // _L):
            pos_c[p, pl.ds(t * _L, _L)] = pos_v[pl.ds(c * _CHUNK + t * _L, _L)]
        pltpu.async_copy(bufs[p], out_hbm.at[pos_c.at[p]], sems_s[p])

    def wait_scatter(c, p):
        pltpu.make_async_copy(bufs[p], out_hbm.at[pos_c.at[p]], sems_s[p]).wait()

    for u in range(_NBUF - 1):
        issue_gather(jnp.int32(u), u)

    def super_step(c4, carry):
        for u in range(_NBUF):
            t = c4 * _NBUF + u
            pw = (u + _NBUF - 1) % _NBUF

            if u == 0:
                @pl.when(c4 >= 1)
                def _():
                    wait_scatter(t - 1, pw)
            else:
                wait_scatter(t - 1, pw)

            @pl.when(t + _NBUF - 1 < _NCHUNK)
            def _():
                issue_gather(t + _NBUF - 1, pw)

            wait_gather(t, u)
            issue_scatter(t, u)
        return carry

    lax.fori_loop(0, _NCHUNK // _NBUF, super_step, 0)
    wait_scatter(_NCHUNK - 1, (_NCHUNK - 1) % _NBUF)


_lookup = functools.partial(
    pl.kernel,
    out_type=jax.ShapeDtypeStruct((_N, _D), jnp.float32),
    mesh=plsc.VectorSubcoreMesh(core_axis_name="c", subcore_axis_name="s"),
    compiler_params=pltpu.CompilerParams(needs_layout_passes=False, use_tc_tiling_on_sc=False),
    scratch_types=[
        pltpu.VMEM((_PER_W,), jnp.int32),
        pltpu.VMEM((_PER_W + _L,), jnp.int32),
        pltpu.VMEM((_PER_W + _L,), jnp.int32),
        pltpu.VMEM((_NBUF, _CHUNK), jnp.int32),
        pltpu.VMEM((_CHUNK, _D), jnp.float32),
        pltpu.VMEM((_CHUNK, _D), jnp.float32),
        pltpu.VMEM((_CHUNK, _D), jnp.float32),
        pltpu.VMEM((_CHUNK, _D), jnp.float32),
        pltpu.VMEM((_CHUNK, _D), jnp.float32),
        pltpu.VMEM((_L,), jnp.int32),
        pltpu.SemaphoreType.DMA,
        pltpu.SemaphoreType.DMA,
        pltpu.SemaphoreType.DMA,
        pltpu.SemaphoreType.DMA,
        pltpu.SemaphoreType.DMA,
        pltpu.SemaphoreType.DMA,
        pltpu.SemaphoreType.DMA,
        pltpu.SemaphoreType.DMA,
        pltpu.SemaphoreType.DMA,
    ],
)(_body)


def kernel(x, W1, W2):
    xf = x.reshape(-1).astype(jnp.int32)
    out = _lookup(xf, W1, W2)
    return out.reshape(x.shape + (_D,))
